# baseline (speedup 1.0000x reference)
import jax
import jax.numpy as jnp
from jax import lax
from jax.experimental import pallas as pl
from jax.experimental.pallas import tpu as pltpu

_H, _W, _C = 495, 436, 8
_HW = _W * _C
_OFFSETS = (12, 13, 14, 17, 20, 23)
_BH = 99


def _body(idx_ref, ts_ref, stats_ref, out_ref):
    i = pl.program_id(0)
    flat = idx_ref[i]
    w = flat // 32
    yt = lax.rem(flat, 32)
    r7 = lax.broadcasted_iota(jnp.int32, (7, 32), 0)
    c32 = lax.broadcasted_iota(jnp.int32, (7, 32), 1)
    sel = (r7 == w) & (c32 == yt)
    v = jnp.sum(jnp.where(sel, ts_ref[0], 0.0))
    s = jnp.sum(jnp.where(sel, ts_ref[1], 0.0))
    lane = lax.broadcasted_iota(jnp.int32, (1, 1, _BH, _HW), 3)
    scale = jnp.where(lane % 2 == 0, v, s)
    prod = stats_ref[...] * scale
    out_ref[...] = prod.astype(jnp.int32).astype(jnp.float32)


def kernel(x, additional_data, stats, timeshift_arr):
    del x
    b = additional_data.shape[0]
    weekday = additional_data[:, 0]
    time = additional_data[:, 1]
    offs = jnp.asarray(_OFFSETS, dtype=additional_data.dtype)
    y_times = time[:, None] + offs[None, :]
    flat_rows = (weekday[:, None] * 32 + y_times).reshape(-1)
    n = b * 6

    nr = _H // _BH
    stats4 = stats.reshape(7 * 32, nr, _BH, _HW)

    grid_spec = pltpu.PrefetchScalarGridSpec(
        num_scalar_prefetch=1,
        grid=(n, nr),
        in_specs=[
            pl.BlockSpec((2, 7, 32), lambda i, r, idx: (0, 0, 0)),
            pl.BlockSpec((1, 1, _BH, _HW), lambda i, r, idx: (idx[i], r, 0, 0)),
        ],
        out_specs=pl.BlockSpec((1, 1, _BH, _HW), lambda i, r, idx: (i, r, 0, 0)),
    )
    out = pl.pallas_call(
        _body,
        grid_spec=grid_spec,
        out_shape=jax.ShapeDtypeStruct((n, nr, _BH, _HW), jnp.float32),
    )(flat_rows.astype(jnp.int32), timeshift_arr, stats4)
    return out.reshape(b, 6, _H, _W, _C)


# speedup vs baseline: 38.4783x; 38.4783x over previous
import jax
import jax.numpy as jnp
from jax import lax
from jax.experimental import pallas as pl
from jax.experimental.pallas import tpu as pltpu

_H, _W, _C = 495, 436, 8
_WC = _W * _C
_SB = 872
_OFFSETS = (12, 13, 14, 17, 20, 23)


def _body(idx_ref, ts_ref, stats_ref, out_ref):
    i = pl.program_id(0)
    flat = idx_ref[i]
    w = flat // 32
    yt = lax.rem(flat, 32)
    r7 = lax.broadcasted_iota(jnp.int32, (7, 32), 0)
    c32 = lax.broadcasted_iota(jnp.int32, (7, 32), 1)
    sel = (r7 == w) & (c32 == yt)
    v = jnp.sum(jnp.where(sel, ts_ref[0], 0.0))
    s = jnp.sum(jnp.where(sel, ts_ref[1], 0.0))
    sub = lax.broadcasted_iota(jnp.int32, (1, _SB, _H), 1)
    scale = jnp.where(sub % 2 == 0, v, s)
    prod = stats_ref[...] * scale
    out_ref[...] = prod.astype(jnp.int32).astype(jnp.float32)


def kernel(x, additional_data, stats, timeshift_arr):
    del x
    b = additional_data.shape[0]
    weekday = additional_data[:, 0]
    time = additional_data[:, 1]
    offs = jnp.asarray(_OFFSETS, dtype=additional_data.dtype)
    y_times = time[:, None] + offs[None, :]
    flat_rows = (weekday[:, None] * 32 + y_times).reshape(-1)
    n = b * 6

    stats_v = stats.transpose(0, 1, 3, 4, 2).reshape(7 * 32, _WC, _H)

    grid_spec = pltpu.PrefetchScalarGridSpec(
        num_scalar_prefetch=1,
        grid=(n, _WC // _SB),
        in_specs=[
            pl.BlockSpec((2, 7, 32), lambda i, r, idx: (0, 0, 0)),
            pl.BlockSpec((1, _SB, _H), lambda i, r, idx: (idx[i], r, 0)),
        ],
        out_specs=pl.BlockSpec((1, _SB, _H), lambda i, r, idx: (i, r, 0)),
    )
    out = pl.pallas_call(
        _body,
        grid_spec=grid_spec,
        out_shape=jax.ShapeDtypeStruct((n, _WC, _H), jnp.float32),
    )(flat_rows.astype(jnp.int32), timeshift_arr, stats_v)
    return out.reshape(b, 6, _W, _C, _H).transpose(0, 1, 4, 2, 3)


# speedup vs baseline: 47.0875x; 1.2237x over previous
import jax
import jax.numpy as jnp
from jax import lax
from jax.experimental import pallas as pl
from jax.experimental.pallas import tpu as pltpu

_H, _W, _C = 495, 436, 8
_WC = _W * _C
_SB = 1744
_OFFSETS = (12, 13, 14, 17, 20, 23)


def _body(idx_ref, ts_ref, stats_ref, out_ref):
    i = pl.program_id(0)
    flat = idx_ref[i]
    w = flat // 32
    yt = lax.rem(flat, 32)
    r7 = lax.broadcasted_iota(jnp.int32, (7, 32), 0)
    c32 = lax.broadcasted_iota(jnp.int32, (7, 32), 1)
    sel = (r7 == w) & (c32 == yt)
    v = jnp.sum(jnp.where(sel, ts_ref[0], 0.0))
    s = jnp.sum(jnp.where(sel, ts_ref[1], 0.0))
    sub = lax.broadcasted_iota(jnp.int32, (1, _SB, _H), 1)
    scale = jnp.where(sub % 2 == 0, v, s)
    prod = stats_ref[...] * scale
    out_ref[...] = prod.astype(jnp.int32).astype(jnp.float32)


def kernel(x, additional_data, stats, timeshift_arr):
    del x
    b = additional_data.shape[0]
    weekday = additional_data[:, 0]
    time = additional_data[:, 1]
    offs = jnp.asarray(_OFFSETS, dtype=additional_data.dtype)
    y_times = time[:, None] + offs[None, :]
    flat_rows = (weekday[:, None] * 32 + y_times).reshape(-1)
    n = b * 6

    stats_v = stats.transpose(0, 1, 3, 4, 2).reshape(7 * 32, _WC, _H)

    grid_spec = pltpu.PrefetchScalarGridSpec(
        num_scalar_prefetch=1,
        grid=(n, _WC // _SB),
        in_specs=[
            pl.BlockSpec((2, 7, 32), lambda i, r, idx: (0, 0, 0)),
            pl.BlockSpec((1, _SB, _H), lambda i, r, idx: (idx[i], r, 0)),
        ],
        out_specs=pl.BlockSpec((1, _SB, _H), lambda i, r, idx: (i, r, 0)),
    )
    out = pl.pallas_call(
        _body,
        grid_spec=grid_spec,
        out_shape=jax.ShapeDtypeStruct((n, _WC, _H), jnp.float32),
    )(flat_rows.astype(jnp.int32), timeshift_arr, stats_v)
    return out.reshape(b, 6, _W, _C, _H).transpose(0, 1, 4, 2, 3)


# speedup vs baseline: 49.4184x; 1.0495x over previous
import jax
import jax.numpy as jnp
from jax import lax
from jax.experimental import pallas as pl
from jax.experimental.pallas import tpu as pltpu

_H, _W, _C = 495, 436, 8
_WC = _W * _C
_SB = 3488
_OFFSETS = (12, 13, 14, 17, 20, 23)


def _body(idx_ref, ts_ref, stats_ref, out_ref):
    i = pl.program_id(0)
    flat = idx_ref[i]
    w = flat // 32
    yt = lax.rem(flat, 32)
    r7 = lax.broadcasted_iota(jnp.int32, (7, 32), 0)
    c32 = lax.broadcasted_iota(jnp.int32, (7, 32), 1)
    sel = (r7 == w) & (c32 == yt)
    v = jnp.sum(jnp.where(sel, ts_ref[0], 0.0))
    s = jnp.sum(jnp.where(sel, ts_ref[1], 0.0))
    sub = lax.broadcasted_iota(jnp.int32, (1, _SB, _H), 1)
    scale = jnp.where(sub % 2 == 0, v, s)
    prod = stats_ref[...] * scale
    out_ref[...] = prod.astype(jnp.int32).astype(jnp.float32)


def kernel(x, additional_data, stats, timeshift_arr):
    del x
    b = additional_data.shape[0]
    weekday = additional_data[:, 0]
    time = additional_data[:, 1]
    offs = jnp.asarray(_OFFSETS, dtype=additional_data.dtype)
    y_times = time[:, None] + offs[None, :]
    flat_rows = (weekday[:, None] * 32 + y_times).reshape(-1)
    n = b * 6

    stats_v = stats.transpose(0, 1, 3, 4, 2).reshape(7 * 32, _WC, _H)

    grid_spec = pltpu.PrefetchScalarGridSpec(
        num_scalar_prefetch=1,
        grid=(n, _WC // _SB),
        in_specs=[
            pl.BlockSpec((2, 7, 32), lambda i, r, idx: (0, 0, 0)),
            pl.BlockSpec((1, _SB, _H), lambda i, r, idx: (idx[i], r, 0)),
        ],
        out_specs=pl.BlockSpec((1, _SB, _H), lambda i, r, idx: (i, r, 0)),
    )
    out = pl.pallas_call(
        _body,
        grid_spec=grid_spec,
        out_shape=jax.ShapeDtypeStruct((n, _WC, _H), jnp.float32),
    )(flat_rows.astype(jnp.int32), timeshift_arr, stats_v)
    return out.reshape(b, 6, _W, _C, _H).transpose(0, 1, 4, 2, 3)
